# baseline (speedup 1.0000x reference)
import functools

import jax
import jax.numpy as jnp
from jax import lax
from jax.experimental import pallas as pl
from jax.experimental.pallas import tpu as pltpu
from jax.experimental.pallas import tpu_sc as plsc

B, S = 128, 128
D = 1024
N_DS = 100000
K = 8
LAMBDA_KNN = 0.25
KNN_TEMPERATURE = 10.0

SC_PARAMS = pltpu.CompilerParams(needs_layout_passes=False)

NW = 32
ROWS_PER_W = B // NW
TOK_CHUNK = 64
N_CHUNK = 2000
N_STEPS = N_DS // N_CHUNK
NEG_INF = -3.0e38
I32_BIG = 2**31 - 1


def _pool_body(emb_hbm, ids_hbm, w_hbm, out_hbm, idx_v, w_v, rows_v, acc_v, sem):
    wid = lax.axis_index("s") * 2 + lax.axis_index("c")
    for r in range(ROWS_PER_W):
        b = wid * ROWS_PER_W + r
        for c2 in range(S // TOK_CHUNK):
            base = b * S + c2 * TOK_CHUNK
            pltpu.sync_copy(ids_hbm.at[pl.ds(base, TOK_CHUNK)], idx_v)
            pltpu.sync_copy(w_hbm.at[pl.ds(base, TOK_CHUNK)], w_v)
            pltpu.async_copy(emb_hbm.at[idx_v], rows_v, sem).wait()
            ws = []
            for g in range(TOK_CHUNK // 16):
                wvec = w_v[pl.ds(g * 16, 16)]
                ws.extend(wvec[jj] for jj in range(16))
            first = c2 == 0

            def col_body(c, _):
                sl = pl.ds(c * 16, 16)
                accs = [rows_v[j, sl] * ws[j] for j in range(4)]
                for j in range(4, TOK_CHUNK):
                    accs[j % 4] = accs[j % 4] + rows_v[j, sl] * ws[j]
                acc = (accs[0] + accs[1]) + (accs[2] + accs[3])
                if first:
                    acc_v[sl] = acc
                else:
                    acc_v[sl] = acc_v[sl] + acc
                return 0

            lax.fori_loop(0, D // 16, col_body, 0)
        pltpu.sync_copy(acc_v, out_hbm.at[b])


def _pool(word_embedding, ids, w_tok):
    mesh = plsc.VectorSubcoreMesh(core_axis_name="c", subcore_axis_name="s")
    f = pl.kernel(
        _pool_body,
        out_type=jax.ShapeDtypeStruct((B, D), jnp.float32),
        mesh=mesh,
        scratch_types=[
            pltpu.VMEM((TOK_CHUNK,), jnp.int32),
            pltpu.VMEM((TOK_CHUNK,), jnp.float32),
            pltpu.VMEM((TOK_CHUNK, D), jnp.float32),
            pltpu.VMEM((D,), jnp.float32),
            pltpu.SemaphoreType.DMA,
        ],
        compiler_params=SC_PARAMS,
    )
    return f(word_embedding, ids, w_tok)


def _topk_body(h_ref, k_ref, idx_ref, w_ref, tv, ti):
    i = pl.program_id(0)

    @pl.when(i == 0)
    def _():
        tv[...] = jnp.full((B, 2 * K), NEG_INF, jnp.float32)
        ti[...] = jnp.zeros((B, 2 * K), jnp.int32)

    h = h_ref[...]
    k = k_ref[...]
    s = 2.0 * lax.dot_general(
        h, k, (((1,), (1,)), ((), ())),
        precision=lax.Precision.HIGHEST,
        preferred_element_type=jnp.float32)
    ksq = jnp.sum(k * k, axis=1)
    s = s - ksq[None, :]

    cols = lax.broadcasted_iota(jnp.int32, (B, N_CHUNK), 1)
    gidx = i * N_CHUNK + cols
    for t in range(K):
        m = jnp.max(s, axis=1, keepdims=True)
        cand = jnp.where(s >= m, gidx, I32_BIG)
        j = jnp.min(cand, axis=1, keepdims=True)
        tv[:, K + t] = m[:, 0]
        ti[:, K + t] = j[:, 0]
        s = jnp.where(gidx == j, NEG_INF, s)

    v = tv[...]
    ix = ti[...]
    cols16 = lax.broadcasted_iota(jnp.int32, (B, 2 * K), 1)
    new_v, new_i = [], []
    for t in range(K):
        m = jnp.max(v, axis=1, keepdims=True)
        cand = jnp.where(v >= m, cols16, I32_BIG)
        j = jnp.min(cand, axis=1, keepdims=True)
        sel = cols16 == j
        gi = jnp.sum(jnp.where(sel, ix, 0), axis=1, keepdims=True)
        new_v.append(m)
        new_i.append(gi)
        v = jnp.where(sel, NEG_INF, v)
    merged_v = jnp.concatenate(new_v, axis=1)
    merged_i = jnp.concatenate(new_i, axis=1)
    tv[:, :K] = merged_v
    ti[:, :K] = merged_i

    @pl.when(i == N_STEPS - 1)
    def _():
        e = jnp.exp((merged_v - merged_v[:, :1]) / KNN_TEMPERATURE)
        w_ref[...] = e / jnp.sum(e, axis=1, keepdims=True)
        idx_ref[...] = merged_i


def _topk(hidden, keys):
    return pl.pallas_call(
        _topk_body,
        grid=(N_STEPS,),
        in_specs=[
            pl.BlockSpec((B, D), lambda i: (0, 0)),
            pl.BlockSpec((N_CHUNK, D), lambda i: (i, 0)),
        ],
        out_specs=[
            pl.BlockSpec((B, K), lambda i: (0, 0)),
            pl.BlockSpec((B, K), lambda i: (0, 0)),
        ],
        out_shape=[
            jax.ShapeDtypeStruct((B, K), jnp.int32),
            jax.ShapeDtypeStruct((B, K), jnp.float32),
        ],
        scratch_shapes=[
            pltpu.VMEM((B, 2 * K), jnp.float32),
            pltpu.VMEM((B, 2 * K), jnp.int32),
        ],
        compiler_params=pltpu.CompilerParams(
            dimension_semantics=("arbitrary",)),
    )(hidden, keys)


def _finish_body(hid_hbm, val_hbm, idx_hbm, w_hbm, cw_hbm, cb_hbm, out_hbm,
                 h_v, rows_v, idx_v, w_v, cw_v, cb_v, out_v, sem):
    wid = lax.axis_index("s") * 2 + lax.axis_index("c")
    nidx = ROWS_PER_W * K
    pltpu.sync_copy(cw_hbm, cw_v)
    pltpu.sync_copy(cb_hbm, cb_v)
    pltpu.sync_copy(idx_hbm.at[pl.ds(wid * nidx, nidx)], idx_v)
    pltpu.sync_copy(w_hbm.at[pl.ds(wid * nidx, nidx)], w_v)
    pltpu.async_copy(val_hbm.at[idx_v], rows_v, sem).wait()
    ws = []
    for g in range(nidx // 16):
        wvec = w_v[pl.ds(g * 16, 16)]
        ws.extend(wvec[jj] for jj in range(16))
    cbvec = cb_v[...]
    cb0, cb1 = cbvec[0], cbvec[1]
    for r in range(ROWS_PER_W):
        b = wid * ROWS_PER_W + r
        pltpu.sync_copy(hid_hbm.at[b], h_v)

        def body(c, carry):
            l0, l1 = carry
            sl = pl.ds(c * 16, 16)
            acc = rows_v[r * K, sl] * ws[r * K]
            for j in range(1, K):
                acc = acc + rows_v[r * K + j, sl] * ws[r * K + j]
            last = (1.0 - LAMBDA_KNN) * h_v[sl] + LAMBDA_KNN * acc
            return (l0 + last * cw_v[0, sl], l1 + last * cw_v[1, sl])

        zero = jnp.zeros((16,), jnp.float32)
        l0, l1 = lax.fori_loop(0, D // 16, body, (zero, zero))
        s0 = jnp.sum(l0) + cb0
        s1 = jnp.sum(l1) + cb1
        lane = lax.iota(jnp.int32, 16)
        out_v[...] = jnp.where(lane == 0, s0, jnp.where(lane == 1, s1, 0.0))
        pltpu.sync_copy(out_v, out_hbm.at[b])


def _finish(hidden, values, idx_flat, w_flat, cw2, cb16):
    mesh = plsc.VectorSubcoreMesh(core_axis_name="c", subcore_axis_name="s")
    f = pl.kernel(
        _finish_body,
        out_type=jax.ShapeDtypeStruct((B, 16), jnp.float32),
        mesh=mesh,
        scratch_types=[
            pltpu.VMEM((D,), jnp.float32),
            pltpu.VMEM((ROWS_PER_W * K, D), jnp.float32),
            pltpu.VMEM((ROWS_PER_W * K,), jnp.int32),
            pltpu.VMEM((ROWS_PER_W * K,), jnp.float32),
            pltpu.VMEM((2, D), jnp.float32),
            pltpu.VMEM((16,), jnp.float32),
            pltpu.VMEM((16,), jnp.float32),
            pltpu.SemaphoreType.DMA,
        ],
        compiler_params=SC_PARAMS,
    )
    return f(hidden, values, idx_flat, w_flat, cw2, cb16)


def kernel(input_ids, attention_mask, word_embedding, datastore_keys,
           datastore_values, classifier_w, classifier_b):
    ids = input_ids.reshape(B * S).astype(jnp.int32)
    denom = jnp.clip(jnp.sum(attention_mask, axis=1), 1e-6, None)
    w_tok = (attention_mask / denom[:, None]).reshape(B * S)
    w_tok = w_tok.astype(jnp.float32)

    hidden = _pool(word_embedding, ids, w_tok)
    idx, wts = _topk(hidden, datastore_keys)

    cw2 = jnp.transpose(classifier_w)
    cb16 = jnp.zeros((16,), jnp.float32).at[:2].set(classifier_b)
    out = _finish(hidden, datastore_values, idx.reshape(B * K),
                  wts.reshape(B * K), cw2, cb16)
    return out[:, :2]


# speedup vs baseline: 1.0173x; 1.0173x over previous
import functools

import jax
import jax.numpy as jnp
from jax import lax
from jax.experimental import pallas as pl
from jax.experimental.pallas import tpu as pltpu
from jax.experimental.pallas import tpu_sc as plsc

B, S = 128, 128
D = 1024
N_DS = 100000
K = 8
LAMBDA_KNN = 0.25
KNN_TEMPERATURE = 10.0

SC_PARAMS = pltpu.CompilerParams(needs_layout_passes=False)

NW = 32
ROWS_PER_W = B // NW
TOK_CHUNK = 64
N_CHUNK = 2000
N_STEPS = N_DS // N_CHUNK
NEG_INF = -3.0e38
I32_BIG = 2**31 - 1


def _pool_body(emb_hbm, ids_hbm, w_hbm, out_hbm, idx_v, w_v, rows_v, acc_v, sem):
    wid = lax.axis_index("s") * 2 + lax.axis_index("c")
    for r in range(ROWS_PER_W):
        b = wid * ROWS_PER_W + r
        for c2 in range(S // TOK_CHUNK):
            base = b * S + c2 * TOK_CHUNK
            pltpu.sync_copy(ids_hbm.at[pl.ds(base, TOK_CHUNK)], idx_v)
            pltpu.sync_copy(w_hbm.at[pl.ds(base, TOK_CHUNK)], w_v)
            pltpu.async_copy(emb_hbm.at[idx_v], rows_v, sem).wait()
            ws = []
            for g in range(TOK_CHUNK // 16):
                wvec = w_v[pl.ds(g * 16, 16)]
                ws.extend(wvec[jj] for jj in range(16))
            first = c2 == 0

            def col_body(c, _):
                sl = pl.ds(c * 16, 16)
                accs = [rows_v[j, sl] * ws[j] for j in range(4)]
                for j in range(4, TOK_CHUNK):
                    accs[j % 4] = accs[j % 4] + rows_v[j, sl] * ws[j]
                acc = (accs[0] + accs[1]) + (accs[2] + accs[3])
                if first:
                    acc_v[sl] = acc
                else:
                    acc_v[sl] = acc_v[sl] + acc
                return 0

            lax.fori_loop(0, D // 16, col_body, 0)
        pltpu.sync_copy(acc_v, out_hbm.at[b])


def _pool(word_embedding, ids, w_tok):
    mesh = plsc.VectorSubcoreMesh(core_axis_name="c", subcore_axis_name="s")
    f = pl.kernel(
        _pool_body,
        out_type=jax.ShapeDtypeStruct((B, D), jnp.float32),
        mesh=mesh,
        scratch_types=[
            pltpu.VMEM((TOK_CHUNK,), jnp.int32),
            pltpu.VMEM((TOK_CHUNK,), jnp.float32),
            pltpu.VMEM((TOK_CHUNK, D), jnp.float32),
            pltpu.VMEM((D,), jnp.float32),
            pltpu.SemaphoreType.DMA,
        ],
        compiler_params=SC_PARAMS,
    )
    return f(word_embedding, ids, w_tok)


def _topk_body(h_ref, k_ref, qsq_ref, ksq_ref, idx_ref, w_ref, tv, ti):
    i = pl.program_id(0)

    @pl.when(i == 0)
    def _():
        tv[...] = jnp.full((B, 2 * K), NEG_INF, jnp.float32)
        ti[...] = jnp.zeros((B, 2 * K), jnp.int32)

    h = h_ref[...]
    k = k_ref[...]
    m = lax.dot_general(
        h, k, (((1,), (1,)), ((), ())),
        precision=lax.Precision.HIGHEST,
        preferred_element_type=jnp.float32)
    i_ = pl.program_id(0)
    ksq = ksq_ref[pl.ds(i_, 1), :]
    qsq = qsq_ref[...]
    s = -((qsq - 2.0 * m) + ksq)

    rowmax = jnp.max(s, axis=1)
    thresh = tv[:, K - 1]
    trigger = jnp.any(rowmax > thresh)

    @pl.when(trigger)
    def _():
        sm = s
        cols = lax.broadcasted_iota(jnp.int32, (B, N_CHUNK), 1)
        gidx = i * N_CHUNK + cols
        for t in range(K):
            m = jnp.max(sm, axis=1, keepdims=True)
            cand = jnp.where(sm >= m, gidx, I32_BIG)
            j = jnp.min(cand, axis=1, keepdims=True)
            tv[:, K + t] = m[:, 0]
            ti[:, K + t] = j[:, 0]
            sm = jnp.where(gidx == j, NEG_INF, sm)

        v = tv[...]
        ix = ti[...]
        cols16 = lax.broadcasted_iota(jnp.int32, (B, 2 * K), 1)
        new_v, new_i = [], []
        for t in range(K):
            m = jnp.max(v, axis=1, keepdims=True)
            cand = jnp.where(v >= m, cols16, I32_BIG)
            j = jnp.min(cand, axis=1, keepdims=True)
            sel = cols16 == j
            gi = jnp.sum(jnp.where(sel, ix, 0), axis=1, keepdims=True)
            new_v.append(m)
            new_i.append(gi)
            v = jnp.where(sel, NEG_INF, v)
        tv[:, :K] = jnp.concatenate(new_v, axis=1)
        ti[:, :K] = jnp.concatenate(new_i, axis=1)

    @pl.when(i == N_STEPS - 1)
    def _():
        mv = tv[:, :K] / KNN_TEMPERATURE
        e = jnp.exp(mv - mv[:, :1])
        w_ref[...] = e / jnp.sum(e, axis=1, keepdims=True)
        idx_ref[...] = ti[:, :K]


def _topk(hidden, keys, qsq, ksq):
    return pl.pallas_call(
        _topk_body,
        grid=(N_STEPS,),
        in_specs=[
            pl.BlockSpec((B, D), lambda i: (0, 0)),
            pl.BlockSpec((N_CHUNK, D), lambda i: (i, 0)),
            pl.BlockSpec((B, 1), lambda i: (0, 0)),
            pl.BlockSpec((N_STEPS, N_CHUNK), lambda i: (0, 0)),
        ],
        out_specs=[
            pl.BlockSpec((B, K), lambda i: (0, 0)),
            pl.BlockSpec((B, K), lambda i: (0, 0)),
        ],
        out_shape=[
            jax.ShapeDtypeStruct((B, K), jnp.int32),
            jax.ShapeDtypeStruct((B, K), jnp.float32),
        ],
        scratch_shapes=[
            pltpu.VMEM((B, 2 * K), jnp.float32),
            pltpu.VMEM((B, 2 * K), jnp.int32),
        ],
        compiler_params=pltpu.CompilerParams(
            dimension_semantics=("arbitrary",)),
    )(hidden, keys, qsq, ksq)


def _finish_body(hid_hbm, val_hbm, idx_hbm, w_hbm, cw_hbm, cb_hbm, out_hbm,
                 h_v, rows_v, idx_v, w_v, cw_v, cb_v, out_v, sem):
    wid = lax.axis_index("s") * 2 + lax.axis_index("c")
    nidx = ROWS_PER_W * K
    pltpu.sync_copy(cw_hbm, cw_v)
    pltpu.sync_copy(cb_hbm, cb_v)
    pltpu.sync_copy(idx_hbm.at[pl.ds(wid * nidx, nidx)], idx_v)
    pltpu.sync_copy(w_hbm.at[pl.ds(wid * nidx, nidx)], w_v)
    pltpu.async_copy(val_hbm.at[idx_v], rows_v, sem).wait()
    ws = []
    for g in range(nidx // 16):
        wvec = w_v[pl.ds(g * 16, 16)]
        ws.extend(wvec[jj] for jj in range(16))
    cbvec = cb_v[...]
    cb0, cb1 = cbvec[0], cbvec[1]
    for r in range(ROWS_PER_W):
        b = wid * ROWS_PER_W + r
        pltpu.sync_copy(hid_hbm.at[b], h_v)

        def body(c, carry):
            l0, l1 = carry
            sl = pl.ds(c * 16, 16)
            acc = rows_v[r * K, sl] * ws[r * K]
            for j in range(1, K):
                acc = acc + rows_v[r * K + j, sl] * ws[r * K + j]
            last = (1.0 - LAMBDA_KNN) * h_v[sl] + LAMBDA_KNN * acc
            return (l0 + last * cw_v[0, sl], l1 + last * cw_v[1, sl])

        zero = jnp.zeros((16,), jnp.float32)
        l0, l1 = lax.fori_loop(0, D // 16, body, (zero, zero))
        s0 = jnp.sum(l0) + cb0
        s1 = jnp.sum(l1) + cb1
        lane = lax.iota(jnp.int32, 16)
        out_v[...] = jnp.where(lane == 0, s0, jnp.where(lane == 1, s1, 0.0))
        pltpu.sync_copy(out_v, out_hbm.at[b])


def _finish(hidden, values, idx_flat, w_flat, cw2, cb16):
    mesh = plsc.VectorSubcoreMesh(core_axis_name="c", subcore_axis_name="s")
    f = pl.kernel(
        _finish_body,
        out_type=jax.ShapeDtypeStruct((B, 16), jnp.float32),
        mesh=mesh,
        scratch_types=[
            pltpu.VMEM((D,), jnp.float32),
            pltpu.VMEM((ROWS_PER_W * K, D), jnp.float32),
            pltpu.VMEM((ROWS_PER_W * K,), jnp.int32),
            pltpu.VMEM((ROWS_PER_W * K,), jnp.float32),
            pltpu.VMEM((2, D), jnp.float32),
            pltpu.VMEM((16,), jnp.float32),
            pltpu.VMEM((16,), jnp.float32),
            pltpu.SemaphoreType.DMA,
        ],
        compiler_params=SC_PARAMS,
    )
    return f(hidden, values, idx_flat, w_flat, cw2, cb16)


def kernel(input_ids, attention_mask, word_embedding, datastore_keys,
           datastore_values, classifier_w, classifier_b):
    ids = input_ids.reshape(B * S).astype(jnp.int32)
    denom = jnp.clip(jnp.sum(attention_mask, axis=1), 1e-6, None)
    w_tok = (attention_mask / denom[:, None]).reshape(B * S)
    w_tok = w_tok.astype(jnp.float32)

    hidden = _pool(word_embedding, ids, w_tok)

    q_sq = (hidden ** 2).sum(axis=-1, keepdims=True)
    k_sq = (datastore_keys ** 2).sum(axis=-1)
    idx, wts = _topk(hidden, datastore_keys, q_sq,
                     k_sq.reshape(N_STEPS, N_CHUNK))

    out = _finish(hidden, datastore_values,
                  idx.reshape(B * K), wts.reshape(B * K),
                  classifier_w.T, jnp.pad(classifier_b, (0, 14)))
    return out[:, :2]


# speedup vs baseline: 1.7314x; 1.7019x over previous
import functools

import jax
import jax.numpy as jnp
from jax import lax
from jax.experimental import pallas as pl
from jax.experimental.pallas import tpu as pltpu
from jax.experimental.pallas import tpu_sc as plsc

B, S = 128, 128
D = 1024
N_DS = 100000
K = 8
LAMBDA_KNN = 0.25
KNN_TEMPERATURE = 10.0

SC_PARAMS = pltpu.CompilerParams(needs_layout_passes=False)

NW = 32
ROWS_PER_W = B // NW
TOK_CHUNK = 64
N_CHUNK = 2000
N_STEPS = N_DS // N_CHUNK
NEG_INF = -3.0e38
I32_BIG = 2**31 - 1


def _pool_body(emb_hbm, ids_hbm, w_hbm, out_hbm, idx_v, w_v, rows_v, acc_v, sem):
    wid = lax.axis_index("s") * 2 + lax.axis_index("c")
    for r in range(ROWS_PER_W):
        b = wid * ROWS_PER_W + r
        for c2 in range(S // TOK_CHUNK):
            base = b * S + c2 * TOK_CHUNK
            pltpu.sync_copy(ids_hbm.at[pl.ds(base, TOK_CHUNK)], idx_v)
            pltpu.sync_copy(w_hbm.at[pl.ds(base, TOK_CHUNK)], w_v)
            pltpu.async_copy(emb_hbm.at[idx_v], rows_v, sem).wait()
            ws = []
            for g in range(TOK_CHUNK // 16):
                wvec = w_v[pl.ds(g * 16, 16)]
                ws.extend(wvec[jj] for jj in range(16))
            first = c2 == 0

            def col_body(c, _):
                sl = pl.ds(c * 16, 16)
                accs = [rows_v[j, sl] * ws[j] for j in range(4)]
                for j in range(4, TOK_CHUNK):
                    accs[j % 4] = accs[j % 4] + rows_v[j, sl] * ws[j]
                acc = (accs[0] + accs[1]) + (accs[2] + accs[3])
                if first:
                    acc_v[sl] = acc
                else:
                    acc_v[sl] = acc_v[sl] + acc
                return 0

            lax.fori_loop(0, D // 16, col_body, 0)
        pltpu.sync_copy(acc_v, out_hbm.at[b])


def _pool(word_embedding, ids, w_tok):
    mesh = plsc.VectorSubcoreMesh(core_axis_name="c", subcore_axis_name="s")
    f = pl.kernel(
        _pool_body,
        out_type=jax.ShapeDtypeStruct((B, D), jnp.float32),
        mesh=mesh,
        scratch_types=[
            pltpu.VMEM((TOK_CHUNK,), jnp.int32),
            pltpu.VMEM((TOK_CHUNK,), jnp.float32),
            pltpu.VMEM((TOK_CHUNK, D), jnp.float32),
            pltpu.VMEM((D,), jnp.float32),
            pltpu.SemaphoreType.DMA,
        ],
        compiler_params=SC_PARAMS,
    )
    return f(word_embedding, ids, w_tok)


def _topk_body(h_ref, k_ref, qsq_ref, ksq_ref, idx_ref, w_ref, tv, ti):
    i = pl.program_id(0)

    @pl.when(i == 0)
    def _():
        tv[...] = jnp.full((B, 2 * K), NEG_INF, jnp.float32)
        ti[...] = jnp.zeros((B, 2 * K), jnp.int32)

    h = h_ref[...]
    k = k_ref[...]
    m = lax.dot_general(
        h, k, (((1,), (1,)), ((), ())),
        precision=lax.Precision.DEFAULT,
        preferred_element_type=jnp.float32)
    i_ = pl.program_id(0)
    ksq = ksq_ref[pl.ds(i_, 1), :]
    qsq = qsq_ref[...]
    s = -((qsq - 2.0 * m) + ksq)

    rowmax = jnp.max(s, axis=1)
    thresh = tv[:, K - 1]
    trigger = jnp.any(rowmax > thresh)

    @pl.when(trigger)
    def _():
        sm = s
        cols = lax.broadcasted_iota(jnp.int32, (B, N_CHUNK), 1)
        gidx = i * N_CHUNK + cols
        for t in range(K):
            m = jnp.max(sm, axis=1, keepdims=True)
            cand = jnp.where(sm >= m, gidx, I32_BIG)
            j = jnp.min(cand, axis=1, keepdims=True)
            tv[:, K + t] = m[:, 0]
            ti[:, K + t] = j[:, 0]
            sm = jnp.where(gidx == j, NEG_INF, sm)

        v = tv[...]
        ix = ti[...]
        cols16 = lax.broadcasted_iota(jnp.int32, (B, 2 * K), 1)
        new_v, new_i = [], []
        for t in range(K):
            m = jnp.max(v, axis=1, keepdims=True)
            cand = jnp.where(v >= m, cols16, I32_BIG)
            j = jnp.min(cand, axis=1, keepdims=True)
            sel = cols16 == j
            gi = jnp.sum(jnp.where(sel, ix, 0), axis=1, keepdims=True)
            new_v.append(m)
            new_i.append(gi)
            v = jnp.where(sel, NEG_INF, v)
        tv[:, :K] = jnp.concatenate(new_v, axis=1)
        ti[:, :K] = jnp.concatenate(new_i, axis=1)

    @pl.when(i == N_STEPS - 1)
    def _():
        mv = tv[:, :K] / KNN_TEMPERATURE
        e = jnp.exp(mv - mv[:, :1])
        w_ref[...] = e / jnp.sum(e, axis=1, keepdims=True)
        idx_ref[...] = ti[:, :K]


def _topk(hidden, keys, qsq, ksq):
    return pl.pallas_call(
        _topk_body,
        grid=(N_STEPS,),
        in_specs=[
            pl.BlockSpec((B, D), lambda i: (0, 0)),
            pl.BlockSpec((N_CHUNK, D), lambda i: (i, 0)),
            pl.BlockSpec((B, 1), lambda i: (0, 0)),
            pl.BlockSpec((N_STEPS, N_CHUNK), lambda i: (0, 0)),
        ],
        out_specs=[
            pl.BlockSpec((B, K), lambda i: (0, 0)),
            pl.BlockSpec((B, K), lambda i: (0, 0)),
        ],
        out_shape=[
            jax.ShapeDtypeStruct((B, K), jnp.int32),
            jax.ShapeDtypeStruct((B, K), jnp.float32),
        ],
        scratch_shapes=[
            pltpu.VMEM((B, 2 * K), jnp.float32),
            pltpu.VMEM((B, 2 * K), jnp.int32),
        ],
        compiler_params=pltpu.CompilerParams(
            dimension_semantics=("arbitrary",)),
    )(hidden, keys, qsq, ksq)


def _finish_body(hid_hbm, val_hbm, idx_hbm, w_hbm, cw_hbm, cb_hbm, out_hbm,
                 h_v, rows_v, idx_v, w_v, cw_v, cb_v, out_v, sem):
    wid = lax.axis_index("s") * 2 + lax.axis_index("c")
    nidx = ROWS_PER_W * K
    pltpu.sync_copy(cw_hbm, cw_v)
    pltpu.sync_copy(cb_hbm, cb_v)
    pltpu.sync_copy(idx_hbm.at[pl.ds(wid * nidx, nidx)], idx_v)
    pltpu.sync_copy(w_hbm.at[pl.ds(wid * nidx, nidx)], w_v)
    pltpu.async_copy(val_hbm.at[idx_v], rows_v, sem).wait()
    ws = []
    for g in range(nidx // 16):
        wvec = w_v[pl.ds(g * 16, 16)]
        ws.extend(wvec[jj] for jj in range(16))
    cbvec = cb_v[...]
    cb0, cb1 = cbvec[0], cbvec[1]
    for r in range(ROWS_PER_W):
        b = wid * ROWS_PER_W + r
        pltpu.sync_copy(hid_hbm.at[b], h_v)

        def body(c, carry):
            l0, l1 = carry
            sl = pl.ds(c * 16, 16)
            acc = rows_v[r * K, sl] * ws[r * K]
            for j in range(1, K):
                acc = acc + rows_v[r * K + j, sl] * ws[r * K + j]
            last = (1.0 - LAMBDA_KNN) * h_v[sl] + LAMBDA_KNN * acc
            return (l0 + last * cw_v[0, sl], l1 + last * cw_v[1, sl])

        zero = jnp.zeros((16,), jnp.float32)
        l0, l1 = lax.fori_loop(0, D // 16, body, (zero, zero))
        s0 = jnp.sum(l0) + cb0
        s1 = jnp.sum(l1) + cb1
        lane = lax.iota(jnp.int32, 16)
        out_v[...] = jnp.where(lane == 0, s0, jnp.where(lane == 1, s1, 0.0))
        pltpu.sync_copy(out_v, out_hbm.at[b])


def _finish(hidden, values, idx_flat, w_flat, cw2, cb16):
    mesh = plsc.VectorSubcoreMesh(core_axis_name="c", subcore_axis_name="s")
    f = pl.kernel(
        _finish_body,
        out_type=jax.ShapeDtypeStruct((B, 16), jnp.float32),
        mesh=mesh,
        scratch_types=[
            pltpu.VMEM((D,), jnp.float32),
            pltpu.VMEM((ROWS_PER_W * K, D), jnp.float32),
            pltpu.VMEM((ROWS_PER_W * K,), jnp.int32),
            pltpu.VMEM((ROWS_PER_W * K,), jnp.float32),
            pltpu.VMEM((2, D), jnp.float32),
            pltpu.VMEM((16,), jnp.float32),
            pltpu.VMEM((16,), jnp.float32),
            pltpu.SemaphoreType.DMA,
        ],
        compiler_params=SC_PARAMS,
    )
    return f(hidden, values, idx_flat, w_flat, cw2, cb16)


def kernel(input_ids, attention_mask, word_embedding, datastore_keys,
           datastore_values, classifier_w, classifier_b):
    ids = input_ids.reshape(B * S).astype(jnp.int32)
    denom = jnp.clip(jnp.sum(attention_mask, axis=1), 1e-6, None)
    w_tok = (attention_mask / denom[:, None]).reshape(B * S)
    w_tok = w_tok.astype(jnp.float32)

    hidden = _pool(word_embedding, ids, w_tok)

    q_sq = (hidden ** 2).sum(axis=-1, keepdims=True)
    k_sq = (datastore_keys ** 2).sum(axis=-1)
    idx, wts = _topk(hidden, datastore_keys, q_sq,
                     k_sq.reshape(N_STEPS, N_CHUNK))

    out = _finish(hidden, datastore_values,
                  idx.reshape(B * K), wts.reshape(B * K),
                  classifier_w.T, jnp.pad(classifier_b, (0, 14)))
    return out[:, :2]
